# baseline (speedup 1.0000x reference)
import functools

import jax
import jax.numpy as jnp
from jax import lax
from jax.experimental import pallas as pl
from jax.experimental.pallas import tpu as pltpu
from jax.experimental.pallas import tpu_sc as plsc

NUM_SPEAKERS = 100000
EMBED_DIM = 64
BATCH = 16384

NUM_CORES = 2
NUM_SUBCORES = 16
NUM_WORKERS = NUM_CORES * NUM_SUBCORES
ROWS_PER_W = EMBED_DIM // NUM_WORKERS
LANES = 16
BCHUNK = 4096
NCHUNK = BATCH // BCHUNK
UNROLL = 8


def _make_gather():
    mesh = plsc.VectorSubcoreMesh(core_axis_name="c", subcore_axis_name="s")

    @functools.partial(
        pl.kernel,
        mesh=mesh,
        out_type=jax.ShapeDtypeStruct((EMBED_DIM, BATCH), jnp.float32),
        scratch_types=[
            pltpu.VMEM((NUM_SPEAKERS,), jnp.float32),
            pltpu.VMEM((BATCH,), jnp.int32),
            pltpu.VMEM((2 * BCHUNK,), jnp.float32),
            pltpu.SemaphoreType.DMA,
            pltpu.SemaphoreType.DMA,
            [pltpu.SemaphoreType.DMA] * 2,
        ],
        compiler_params=pltpu.CompilerParams(
            use_tc_tiling_on_sc=False, needs_layout_passes=False
        ),
    )
    def gather(t2d_hbm, idx_hbm, out_hbm, row_v, idx_v, outc_v,
               rsem, isem, osems):
        wid = lax.axis_index("s") * NUM_CORES + lax.axis_index("c")

        def slot(r):
            return pl.ds(r * BCHUNK, BCHUNK)

        def wait_out(d, c, r):
            pltpu.make_async_copy(
                outc_v.at[slot(r)],
                out_hbm.at[d, pl.ds(c * BCHUNK, BCHUNK)],
                osems[r],
            ).wait()

        icopy = pltpu.async_copy(idx_hbm, idx_v, isem)
        first = pltpu.async_copy(
            t2d_hbm.at[wid * ROWS_PER_W], row_v, rsem
        )
        icopy.wait()
        first.wait()

        def do_row(i, carry):
            d = wid * ROWS_PER_W + i

            def do_chunk(c, r):
                @pl.when(c > 1)
                def _():
                    wait_out(d, c - 2, r)

                def gather16(j, carry3):
                    for u in range(UNROLL):
                        o = c * BCHUNK + (j * UNROLL + u) * LANES
                        iv = idx_v[pl.ds(o, LANES)]
                        v = plsc.load_gather(row_v, [iv])
                        outc_v[pl.ds(r * BCHUNK + (j * UNROLL + u) * LANES,
                                     LANES)] = v
                    return carry3

                lax.fori_loop(0, BCHUNK // LANES // UNROLL, gather16, 0)

                pltpu.async_copy(
                    outc_v.at[slot(r)],
                    out_hbm.at[d, pl.ds(c * BCHUNK, BCHUNK)],
                    osems[r],
                )

            def do_chunk_pair(kk, carry2):
                do_chunk(kk * 2, 0)
                do_chunk(kk * 2 + 1, 1)
                return carry2

            lax.fori_loop(0, NCHUNK // 2, do_chunk_pair, 0)

            wait_out(d, NCHUNK - 2, 0)

            @pl.when(i < ROWS_PER_W - 1)
            def _():
                pltpu.async_copy(t2d_hbm.at[d + 1], row_v, rsem)

            wait_out(d, NCHUNK - 1, 1)

            @pl.when(i < ROWS_PER_W - 1)
            def _():
                pltpu.make_async_copy(
                    t2d_hbm.at[d + 1], row_v, rsem
                ).wait()

            return carry

        lax.fori_loop(0, ROWS_PER_W, do_row, 0)

    return gather


_gather = _make_gather()


@jax.jit
def kernel(table, spk_id):
    out_t = _gather(table.T, spk_id.astype(jnp.int32))
    return out_t.T[:, :, None]


# speedup vs baseline: 1.0646x; 1.0646x over previous
import functools

import jax
import jax.numpy as jnp
from jax import lax
from jax.experimental import pallas as pl
from jax.experimental.pallas import tpu as pltpu
from jax.experimental.pallas import tpu_sc as plsc

NUM_SPEAKERS = 100000
EMBED_DIM = 64
BATCH = 16384

NUM_CORES = 2
NUM_SUBCORES = 16
NUM_WORKERS = NUM_CORES * NUM_SUBCORES
ROWS_PER_W = EMBED_DIM // NUM_WORKERS
LANES = 16
BCHUNK = 4096
NCHUNK = BATCH // BCHUNK


def _make_gather():
    mesh = plsc.VectorSubcoreMesh(core_axis_name="c", subcore_axis_name="s")

    @functools.partial(
        pl.kernel,
        mesh=mesh,
        out_type=jax.ShapeDtypeStruct((EMBED_DIM, BATCH), jnp.float32),
        scratch_types=[
            pltpu.VMEM((NUM_SPEAKERS,), jnp.float32),
            pltpu.VMEM((2 * BCHUNK,), jnp.int32),
            pltpu.VMEM((2 * BCHUNK,), jnp.float32),
            pltpu.SemaphoreType.DMA,
            [pltpu.SemaphoreType.DMA] * 2,
            [pltpu.SemaphoreType.DMA] * 2,
        ],
        compiler_params=pltpu.CompilerParams(
            use_tc_tiling_on_sc=False, needs_layout_passes=False
        ),
    )
    def gather(t2d_hbm, idx_hbm, out_hbm, row_v, idx_v, outc_v,
               rsem, isems, osems):
        wid = lax.axis_index("s") * NUM_CORES + lax.axis_index("c")

        def slot(r):
            return pl.ds(r * BCHUNK, BCHUNK)

        def fire_idx(c, r):
            return pltpu.async_copy(
                idx_hbm.at[pl.ds(c * BCHUNK, BCHUNK)], idx_v.at[slot(r)],
                isems[r],
            )

        def wait_idx(c, r):
            pltpu.make_async_copy(
                idx_hbm.at[pl.ds(c * BCHUNK, BCHUNK)], idx_v.at[slot(r)],
                isems[r],
            ).wait()

        def wait_out(d, c, r):
            pltpu.make_async_copy(
                outc_v.at[slot(r)],
                out_hbm.at[d, pl.ds(c * BCHUNK, BCHUNK)],
                osems[r],
            ).wait()

        fire_idx(0, 0)
        fire_idx(1, 1)

        def do_row(i, carry):
            d = wid * ROWS_PER_W + i
            pltpu.async_copy(t2d_hbm.at[d], row_v, rsem).wait()

            def do_chunk(c, r):
                @pl.when(c > 1)
                def _():
                    wait_out(d, c - 2, r)

                wait_idx(c, r)

                def gather16(j, carry3):
                    for u in range(8):
                        o = (j * 8 + u) * LANES
                        iv = idx_v[pl.ds(r * BCHUNK + o, LANES)]
                        v = plsc.load_gather(row_v, [iv])
                        outc_v[pl.ds(r * BCHUNK + o, LANES)] = v
                    return carry3

                lax.fori_loop(0, BCHUNK // LANES // 8, gather16, 0)

                pltpu.async_copy(
                    outc_v.at[slot(r)],
                    out_hbm.at[d, pl.ds(c * BCHUNK, BCHUNK)],
                    osems[r],
                )

                nxt = c + 2
                is_last_row = i == ROWS_PER_W - 1

                @pl.when(jnp.logical_or(nxt < NCHUNK,
                                        jnp.logical_not(is_last_row)))
                def _():
                    fire_idx(lax.rem(nxt, NCHUNK), r)

            def do_chunk_pair(kk, carry2):
                do_chunk(kk * 2, 0)
                do_chunk(kk * 2 + 1, 1)
                return carry2

            lax.fori_loop(0, NCHUNK // 2, do_chunk_pair, 0)

            wait_out(d, NCHUNK - 2, 0)
            wait_out(d, NCHUNK - 1, 1)
            return carry

        lax.fori_loop(0, ROWS_PER_W, do_row, 0)

    return gather


_gather = _make_gather()


@jax.jit
def kernel(table, spk_id):
    out_t = _gather(table.T, spk_id.astype(jnp.int32))
    return out_t.T[:, :, None]


# speedup vs baseline: 1.0665x; 1.0018x over previous
import functools

import jax
import jax.numpy as jnp
from jax import lax
from jax.experimental import pallas as pl
from jax.experimental.pallas import tpu as pltpu
from jax.experimental.pallas import tpu_sc as plsc

NUM_SPEAKERS = 100000
EMBED_DIM = 64
BATCH = 16384

NUM_CORES = 2
NUM_SUBCORES = 16
NUM_WORKERS = NUM_CORES * NUM_SUBCORES
ROWS_PER_W = EMBED_DIM // NUM_WORKERS
LANES = 16
BCHUNK = 4096
NCHUNK = BATCH // BCHUNK


def _make_gather():
    mesh = plsc.VectorSubcoreMesh(core_axis_name="c", subcore_axis_name="s")

    @functools.partial(
        pl.kernel,
        mesh=mesh,
        out_type=jax.ShapeDtypeStruct((EMBED_DIM, BATCH), jnp.float32),
        scratch_types=[
            pltpu.VMEM((NUM_SPEAKERS,), jnp.float32),
            pltpu.VMEM((2 * BCHUNK,), jnp.int32),
            pltpu.VMEM((2 * BCHUNK,), jnp.float32),
            pltpu.SemaphoreType.DMA,
            [pltpu.SemaphoreType.DMA] * 2,
            [pltpu.SemaphoreType.DMA] * 2,
        ],
        compiler_params=pltpu.CompilerParams(
            use_tc_tiling_on_sc=False, needs_layout_passes=False
        ),
    )
    def gather(t2d_hbm, idx_hbm, out_hbm, row_v, idx_v, outc_v,
               rsem, isems, osems):
        wid = lax.axis_index("s") * NUM_CORES + lax.axis_index("c")

        def slot(r):
            return pl.ds(r * BCHUNK, BCHUNK)

        def fire_idx(c, r):
            return pltpu.async_copy(
                idx_hbm.at[pl.ds(c * BCHUNK, BCHUNK)], idx_v.at[slot(r)],
                isems[r],
            )

        def wait_idx(c, r):
            pltpu.make_async_copy(
                idx_hbm.at[pl.ds(c * BCHUNK, BCHUNK)], idx_v.at[slot(r)],
                isems[r],
            ).wait()

        def wait_out(d, c, r):
            pltpu.make_async_copy(
                outc_v.at[slot(r)],
                out_hbm.at[d, pl.ds(c * BCHUNK, BCHUNK)],
                osems[r],
            ).wait()

        fire_idx(0, 0)
        fire_idx(1, 1)

        def do_row(i, carry):
            d = wid * ROWS_PER_W + i
            pltpu.async_copy(t2d_hbm.at[d], row_v, rsem).wait()

            def do_chunk(c, r):
                @pl.when(c > 1)
                def _():
                    wait_out(d, c - 2, r)

                wait_idx(c, r)

                def gather16(j, carry3):
                    for u in range(4):
                        o = (j * 4 + u) * LANES
                        iv = idx_v[pl.ds(r * BCHUNK + o, LANES)]
                        v = plsc.load_gather(row_v, [iv])
                        outc_v[pl.ds(r * BCHUNK + o, LANES)] = v
                    return carry3

                lax.fori_loop(0, BCHUNK // LANES // 4, gather16, 0)

                pltpu.async_copy(
                    outc_v.at[slot(r)],
                    out_hbm.at[d, pl.ds(c * BCHUNK, BCHUNK)],
                    osems[r],
                )

                nxt = c + 2
                is_last_row = i == ROWS_PER_W - 1

                @pl.when(jnp.logical_or(nxt < NCHUNK,
                                        jnp.logical_not(is_last_row)))
                def _():
                    fire_idx(lax.rem(nxt, NCHUNK), r)

            def do_chunk_pair(kk, carry2):
                do_chunk(kk * 2, 0)
                do_chunk(kk * 2 + 1, 1)
                return carry2

            lax.fori_loop(0, NCHUNK // 2, do_chunk_pair, 0)

            wait_out(d, NCHUNK - 2, 0)
            wait_out(d, NCHUNK - 1, 1)
            return carry

        lax.fori_loop(0, ROWS_PER_W, do_row, 0)

    return gather


_gather = _make_gather()


@jax.jit
def kernel(table, spk_id):
    out_t = _gather(table.T, spk_id.astype(jnp.int32))
    return out_t.T[:, :, None]
